# Initial kernel scaffold; baseline (speedup 1.0000x reference)
import jax
import jax.numpy as jnp
from jax.experimental import pallas as pl


def kernel(mem, idx, val, lookup_idx):
    raise NotImplementedError("write your pallas kernel here")


# speedup vs baseline: 1.5390x; 1.5390x over previous
import functools

import jax
import jax.numpy as jnp
from jax import lax
from jax.experimental import pallas as pl
from jax.experimental.pallas import tpu as pltpu
from jax.experimental.pallas import tpu_sc as plsc

M = 1_000_000
D = 32
B_UPD = 16_384
B_LKP = 4_096
NF = 26

NC, NS, L = 2, 16, 16
NW = NC * NS
CH = 31_264
M_PAD = NW * CH
BPW = B_LKP // NW
LPW = BPW * NF
GB = 4
GRP = GB * NF
NGRP = LPW // GRP

_mesh = plsc.VectorSubcoreMesh(core_axis_name="c", subcore_axis_name="s")
_params = pltpu.CompilerParams(needs_layout_passes=False,
                               use_tc_tiling_on_sc=False)


def _wid():
    return lax.axis_index("s") * NC + lax.axis_index("c")


@functools.partial(
    pl.kernel,
    out_type=jax.ShapeDtypeStruct((M_PAD,), jnp.int32),
    mesh=_mesh,
    scratch_types=[
        pltpu.VMEM((B_UPD,), jnp.int32),
        pltpu.VMEM((CH,), jnp.int32),
    ],
    compiler_params=_params,
)
def _build_owner(idx_hbm, owner_hbm, idx_v, own_v):
    w = _wid()
    lo = w * CH
    pltpu.sync_copy(idx_hbm, idx_v)
    neg = jnp.full((L,), -1, jnp.int32)

    def mset(i, c):
        own_v[pl.ds(i * L, L)] = neg
        return c

    lax.fori_loop(0, CH // L, mset, 0)

    iota = lax.iota(jnp.int32, L)

    def scan(j, c):
        base = j * L
        rel = idx_v[pl.ds(base, L)] - lo
        m = (rel >= 0) & (rel < CH)
        plsc.store_scatter(own_v, [rel], iota + base, mask=m)
        return c

    lax.fori_loop(0, B_UPD // L, scan, 0)
    pltpu.sync_copy(own_v, owner_hbm.at[pl.ds(lo, CH)])


@functools.partial(
    pl.kernel,
    out_type=jax.ShapeDtypeStruct((B_LKP, D), jnp.float32),
    mesh=_mesh,
    scratch_types=[
        pltpu.VMEM((LPW,), jnp.int32),
        pltpu.VMEM((LPW + L,), jnp.int32),
        pltpu.VMEM((LPW,), jnp.int32),
        pltpu.VMEM((2, GRP, D), jnp.float32),
        pltpu.VMEM((2, GRP, D), jnp.float32),
        pltpu.VMEM((BPW, D), jnp.float32),
        pltpu.SemaphoreType.DMA,
        pltpu.SemaphoreType.DMA,
        pltpu.SemaphoreType.DMA,
    ],
    compiler_params=_params,
)
def _lookup(mem_hbm, val_hbm, owner_hbm, lidx_hbm, out_hbm,
            lidx_v, o_v, oc_v, gmem, gval, acc, sem_o, sem_m, sem_v):
    w = _wid()
    pltpu.sync_copy(lidx_hbm.at[pl.ds(w * LPW, LPW)], lidx_v)

    descs = []
    for g in range(NGRP):
        d = pltpu.make_async_copy(
            owner_hbm.at[lidx_v.at[pl.ds(g * GRP, GRP)]],
            o_v.at[pl.ds(g * GRP, GRP)], sem_o)
        d.start()
        descs.append(d)
    for d in descs:
        d.wait()

    def clamp(i, c):
        o = o_v[pl.ds(i * L, L)]
        oc_v[pl.ds(i * L, L)] = jnp.maximum(o, 0)
        return c

    lax.fori_loop(0, LPW // L, clamp, 0)

    def start_group(g, p):
        pltpu.make_async_copy(
            mem_hbm.at[lidx_v.at[pl.ds(g * GRP, GRP)]], gmem.at[p], sem_m
        ).start()
        pltpu.make_async_copy(
            val_hbm.at[oc_v.at[pl.ds(g * GRP, GRP)]], gval.at[p], sem_v
        ).start()

    def wait_group(p):
        pltpu.make_async_copy(
            mem_hbm.at[lidx_v.at[pl.ds(0, GRP)]], gmem.at[p], sem_m).wait()
        pltpu.make_async_copy(
            val_hbm.at[oc_v.at[pl.ds(0, GRP)]], gval.at[p], sem_v).wait()

    start_group(0, 0)
    start_group(1, 1)

    def outer(i, c):
        for p in range(2):
            g = 2 * i + p
            wait_group(p)
            for bl in range(GB):
                alo = jnp.zeros((L,), jnp.float32)
                ahi = jnp.zeros((L,), jnp.float32)
                ov0 = o_v[pl.ds(g * GRP + bl * NF, L)]
                ov1 = o_v[pl.ds(g * GRP + bl * NF + L, L)]
                for f in range(NF):
                    r = bl * NF + f
                    hit = (ov0[f] if f < L else ov1[f - L]) >= 0
                    alo = alo + jnp.where(hit, gval[p, r, pl.ds(0, L)],
                                          gmem[p, r, pl.ds(0, L)])
                    ahi = ahi + jnp.where(hit, gval[p, r, pl.ds(L, L)],
                                          gmem[p, r, pl.ds(L, L)])
                b = g * GB + bl
                acc[b, pl.ds(0, L)] = alo
                acc[b, pl.ds(L, L)] = ahi

            @pl.when(g + 2 < NGRP)
            def _():
                start_group(g + 2, p)
        return c

    lax.fori_loop(0, NGRP // 2, outer, 0)
    pltpu.sync_copy(acc, out_hbm.at[pl.ds(w * BPW, BPW)])


def kernel(mem, idx, val, lookup_idx):
    owner = _build_owner(idx)
    return _lookup(mem, val, owner, lookup_idx.reshape(-1))


# speedup vs baseline: 4.2321x; 2.7498x over previous
import functools

import jax
import jax.numpy as jnp
from jax import lax
from jax.experimental import pallas as pl
from jax.experimental.pallas import tpu as pltpu
from jax.experimental.pallas import tpu_sc as plsc

M = 1_000_000
D = 32
B_UPD = 16_384
B_LKP = 4_096
NF = 26

NC, NS, L = 2, 16, 16
NW = NC * NS
CH = 31_264
M_PAD = NW * CH
BPW = B_LKP // NW
LPW = BPW * NF
GRP = 4 * NF
NGRP = LPW // GRP
SUP = 4
SGR = SUP * GRP
NSG = LPW // SGR
NB = 2
GRPH = 112
NVH = (LPW + GRPH) // L

_mesh = plsc.VectorSubcoreMesh(core_axis_name="c", subcore_axis_name="s")
_params = pltpu.CompilerParams(needs_layout_passes=False,
                               use_tc_tiling_on_sc=False)


def _wid():
    return lax.axis_index("s") * NC + lax.axis_index("c")


@functools.partial(
    pl.kernel,
    out_type=jax.ShapeDtypeStruct((M_PAD,), jnp.int32),
    mesh=_mesh,
    scratch_types=[
        pltpu.VMEM((B_UPD,), jnp.int32),
        pltpu.VMEM((CH,), jnp.int32),
    ],
    compiler_params=_params,
)
def _build_owner(idx_hbm, owner_hbm, idx_v, own_v):
    w = _wid()
    lo = w * CH
    pltpu.sync_copy(idx_hbm, idx_v)
    neg = jnp.full((L,), -1, jnp.int32)

    def mset(i, c):
        own_v[pl.ds(i * L, L)] = neg
        return c

    lax.fori_loop(0, CH // L, mset, 0)

    iota = lax.iota(jnp.int32, L)

    def scan(j, c):
        base = j * L
        rel = idx_v[pl.ds(base, L)] - lo
        m = (rel >= 0) & (rel < CH)
        plsc.store_scatter(own_v, [rel], iota + base, mask=m)
        return c

    lax.fori_loop(0, B_UPD // L, scan, 0)
    pltpu.sync_copy(own_v, owner_hbm.at[pl.ds(lo, CH)])


@functools.partial(
    pl.kernel,
    out_type=jax.ShapeDtypeStruct((B_LKP, D), jnp.float32),
    mesh=_mesh,
    scratch_types=[
        pltpu.VMEM((LPW,), jnp.int32),
        pltpu.VMEM((LPW + L,), jnp.int32),
        pltpu.VMEM((NB, SGR, D), jnp.float32),
        pltpu.VMEM((BPW, D), jnp.float32),
        pltpu.VMEM((NVH * L,), jnp.int32),
        pltpu.VMEM((NVH * L,), jnp.int32),
        pltpu.VMEM((NVH * L,), jnp.int32),
        pltpu.VMEM((GRPH, D), jnp.float32),
        pltpu.VMEM((GRPH, D), jnp.float32),
        pltpu.SemaphoreType.DMA,
        pltpu.SemaphoreType.DMA,
        pltpu.SemaphoreType.DMA,
    ],
    compiler_params=_params,
)
def _lookup(mem_hbm, val_hbm, owner_hbm, lidx_hbm, out_hbm,
            lidx_v, o_v, rbuf, acc, hslot, ho, hlidx, vrow, mrow,
            sem_o, sem_m, sem_h):
    w = _wid()
    pltpu.sync_copy(lidx_hbm.at[pl.ds(w * LPW, LPW)], lidx_v)

    def start_sg(sg, par):
        for k in range(SUP):
            off = sg * SGR + k * GRP
            pltpu.make_async_copy(
                mem_hbm.at[lidx_v.at[pl.ds(off, GRP)]],
                rbuf.at[par, pl.ds(k * GRP, GRP)], sem_m).start()

    def wait_sg(par):
        for k in range(SUP):
            pltpu.make_async_copy(
                mem_hbm.at[lidx_v.at[pl.ds(0, GRP)]],
                rbuf.at[par, pl.ds(k * GRP, GRP)], sem_m).wait()

    start_sg(0, 0)
    start_sg(1, 1)

    descs = []
    for g in range(NGRP):
        d = pltpu.make_async_copy(
            owner_hbm.at[lidx_v.at[pl.ds(g * GRP, GRP)]],
            o_v.at[pl.ds(g * GRP, GRP)], sem_o)
        d.start()
        descs.append(d)
    for d in descs:
        d.wait()

    zero = jnp.zeros((L,), jnp.int32)

    def zinit(i, c):
        ho[pl.ds(i * L, L)] = zero
        hlidx[pl.ds(i * L, L)] = zero
        return c

    lax.fori_loop(0, NVH, zinit, 0)

    iota = lax.iota(jnp.int32, L)

    def comp(i, nh):
        base = i * L
        o16 = o_v[pl.ds(base, L)]
        m = o16 >= 0
        plsc.store_compressed(hslot.at[pl.ds(nh, L)], iota + base, mask=m)
        plsc.store_compressed(ho.at[pl.ds(nh, L)], o16, mask=m)
        plsc.store_compressed(hlidx.at[pl.ds(nh, L)],
                              lidx_v[pl.ds(base, L)], mask=m)
        return nh + plsc.all_reduce_population_count(m)[0]

    nh = lax.fori_loop(0, LPW // L, comp, 0)

    def outer(i, c):
        for par in range(NB):
            sg = NB * i + par
            wait_sg(par)
            for bl in range(SGR // NF):
                alo = jnp.zeros((L,), jnp.float32)
                ahi = jnp.zeros((L,), jnp.float32)
                for f in range(NF):
                    r = bl * NF + f
                    alo = alo + rbuf[par, r, pl.ds(0, L)]
                    ahi = ahi + rbuf[par, r, pl.ds(L, L)]
                b = sg * (SGR // NF) + bl
                acc[b, pl.ds(0, L)] = alo
                acc[b, pl.ds(L, L)] = ahi

            @pl.when(sg + NB < NSG)
            def _():
                start_sg(sg + NB, par)
        return c

    lax.fori_loop(0, NSG // NB, outer, 0)

    nch = (nh + GRPH - 1) // GRPH

    def corr(cc, c):
        off = cc * GRPH
        dv = pltpu.make_async_copy(
            val_hbm.at[ho.at[pl.ds(off, GRPH)]], vrow, sem_h)
        dm = pltpu.make_async_copy(
            mem_hbm.at[hlidx.at[pl.ds(off, GRPH)]], mrow, sem_h)
        dv.start()
        dm.start()
        dv.wait()
        dm.wait()

        def blk(q, c2):
            base = q * L
            s16 = hslot[pl.ds(off + base, L)]
            for kk in range(L):
                @pl.when(off + base + kk < nh)
                def _():
                    b = s16[kk] // NF
                    r = base + kk
                    acc[b, pl.ds(0, L)] = (acc[b, pl.ds(0, L)]
                                           + vrow[r, pl.ds(0, L)]
                                           - mrow[r, pl.ds(0, L)])
                    acc[b, pl.ds(L, L)] = (acc[b, pl.ds(L, L)]
                                           + vrow[r, pl.ds(L, L)]
                                           - mrow[r, pl.ds(L, L)])
            return c2

        lax.fori_loop(0, GRPH // L, blk, 0)
        return c

    lax.fori_loop(0, nch, corr, 0)
    pltpu.sync_copy(acc, out_hbm.at[pl.ds(w * BPW, BPW)])


def kernel(mem, idx, val, lookup_idx):
    owner = _build_owner(idx)
    return _lookup(mem, val, owner, lookup_idx.reshape(-1))
